# Initial kernel scaffold; baseline (speedup 1.0000x reference)
import functools

import jax
import jax.numpy as jnp
from jax import lax
from jax.experimental import pallas as pl
from jax.experimental.pallas import tpu as pltpu
from jax.experimental.pallas import tpu_sc as plsc

N = 10000
D = 128
E = 320000
NPAD = 10240
NC, NS, L = 2, 16, 16
NW = NC * NS
EPW = E // NW
WIN = 80
NWIN = EPW // WIN
RPT = NPAD // NS
BLK = 1024
NBLK = NPAD // BLK


def _proj_body(x_ref, w_ref, asrc_ref, adst_ref, xw_ref, av_ref, bv_ref):
    xw = jnp.dot(x_ref[...], w_ref[...], preferred_element_type=jnp.float32)
    xw_ref[...] = xw
    av_ref[...] = jnp.sum(xw * asrc_ref[...], axis=1, keepdims=True)
    bv_ref[...] = jnp.sum(xw * adst_ref[...], axis=1, keepdims=True)


def _project(x_pad, W, asrc_row, adst_row):
    return pl.pallas_call(
        _proj_body,
        grid=(NBLK,),
        in_specs=[
            pl.BlockSpec((BLK, D), lambda i: (i, 0)),
            pl.BlockSpec((D, D), lambda i: (0, 0)),
            pl.BlockSpec((1, D), lambda i: (0, 0)),
            pl.BlockSpec((1, D), lambda i: (0, 0)),
        ],
        out_specs=[
            pl.BlockSpec((BLK, D), lambda i: (i, 0)),
            pl.BlockSpec((BLK, 1), lambda i: (i, 0)),
            pl.BlockSpec((BLK, 1), lambda i: (i, 0)),
        ],
        out_shape=[
            jax.ShapeDtypeStruct((NPAD, D), jnp.float32),
            jax.ShapeDtypeStruct((NPAD, 1), jnp.float32),
            jax.ShapeDtypeStruct((NPAD, 1), jnp.float32),
        ],
    )(x_pad, W, asrc_row, adst_row)


def _edge_body(xw_hbm, av_hbm, bv_hbm, src_hbm, dst_hbm,
               acc_hbm, den_hbm,
               asrc_t, adst_t, src_t, dst_t, rows_v, e_v, zden_v, acc_sh, den_sh):
    cid = lax.axis_index("c")
    sid = lax.axis_index("s")
    w = cid * NS + sid

    pltpu.sync_copy(av_hbm, asrc_t)
    pltpu.sync_copy(bv_hbm, adst_t)
    pltpu.sync_copy(src_hbm.at[w], src_t)
    pltpu.sync_copy(dst_hbm.at[w], dst_t)

    @pl.loop(0, WIN)
    def _(r):
        for k in range(D // L):
            rows_v[r, pl.ds(k * L, L)] = jnp.zeros((L,), jnp.float32)

    @pl.loop(0, RPT, step=L)
    def _(i):
        zden_v[pl.ds(i, L)] = jnp.zeros((L,), jnp.float32)

    for k in range(RPT // WIN):
        pltpu.sync_copy(rows_v, acc_sh.at[pl.ds(sid * RPT + k * WIN, WIN)])
    pltpu.sync_copy(zden_v, den_sh.at[pl.ds(sid * RPT, RPT)])
    plsc.subcore_barrier()

    @pl.loop(0, NWIN)
    def _(j):
        for g in range(WIN // L):
            sl = pl.ds(g * L, L)
            s_idx = src_t[j, sl]
            d_idx = dst_t[j, sl]
            a_s = plsc.load_gather(asrc_t, [s_idx])
            a_d = plsc.load_gather(adst_t, [d_idx])
            al = a_s + a_d
            al = jnp.maximum(al, 0.2 * al)
            e_v[sl] = jnp.exp(al)

        pltpu.sync_copy(xw_hbm.at[src_t.at[j]], rows_v)

        @pl.loop(0, WIN)
        def _(r):
            bc = plsc.load_gather(e_v, [jnp.full((L,), r, jnp.int32)])
            for k in range(D // L):
                sl = pl.ds(k * L, L)
                rows_v[r, sl] = rows_v[r, sl] * bc

        pltpu.sync_copy(rows_v, acc_sh.at[dst_t.at[j]], add=True)
        pltpu.sync_copy(e_v, den_sh.at[dst_t.at[j]], add=True)

    plsc.subcore_barrier()

    pltpu.sync_copy(acc_sh.at[pl.ds(sid * RPT, RPT)],
                    acc_hbm.at[cid].at[pl.ds(sid * RPT, RPT)])
    pltpu.sync_copy(den_sh.at[pl.ds(sid * RPT, RPT)],
                    den_hbm.at[cid].at[pl.ds(sid * RPT, RPT)])


def _edge_pass(xw, av, bv, src3d, dst3d):
    mesh = plsc.VectorSubcoreMesh(core_axis_name="c", subcore_axis_name="s")
    f = pl.kernel(
        _edge_body,
        mesh=mesh,
        out_type=[
            jax.ShapeDtypeStruct((NC, NPAD, D), jnp.float32),
            jax.ShapeDtypeStruct((NC, NPAD), jnp.float32),
        ],
        scratch_types=[
            pltpu.VMEM((NPAD,), jnp.float32),
            pltpu.VMEM((NPAD,), jnp.float32),
            pltpu.VMEM((NWIN, WIN), jnp.int32),
            pltpu.VMEM((NWIN, WIN), jnp.int32),
            pltpu.VMEM((WIN, D), jnp.float32),
            pltpu.VMEM((WIN,), jnp.float32),
            pltpu.VMEM((RPT,), jnp.float32),
            pltpu.VMEM_SHARED((NPAD, D), jnp.float32),
            pltpu.VMEM_SHARED((NPAD,), jnp.float32),
        ],
    )
    return f(xw, av, bv, src3d, dst3d)


def _final_body(a0_ref, a1_ref, d0_ref, d1_ref, bias_ref, o_ref):
    s = a0_ref[...] + a1_ref[...]
    dd = d0_ref[...] + d1_ref[...] + 1e-16
    v = s / dd + bias_ref[...]
    o_ref[...] = jnp.where(v > 0, v, jnp.expm1(jnp.minimum(v, 0.0)))


def _finalize(acc, den, bias_row):
    return pl.pallas_call(
        _final_body,
        grid=(NBLK,),
        in_specs=[
            pl.BlockSpec((BLK, D), lambda i: (i, 0)),
            pl.BlockSpec((BLK, D), lambda i: (i, 0)),
            pl.BlockSpec((BLK, 1), lambda i: (i, 0)),
            pl.BlockSpec((BLK, 1), lambda i: (i, 0)),
            pl.BlockSpec((1, D), lambda i: (0, 0)),
        ],
        out_specs=pl.BlockSpec((BLK, D), lambda i: (i, 0)),
        out_shape=jax.ShapeDtypeStruct((NPAD, D), jnp.float32),
    )(acc[0], acc[1], den[0][:, None], den[1][:, None], bias_row)


def kernel(x, edge_index, W, att_src, att_dst, bias):
    x_pad = jnp.pad(x, ((0, NPAD - N), (0, 0)))
    asrc_row = att_src.reshape(1, D)
    adst_row = att_dst.reshape(1, D)
    xw, av, bv = _project(x_pad, W, asrc_row, adst_row)
    src3d = edge_index[0].reshape(NW, NWIN, WIN)
    dst3d = edge_index[1].reshape(NW, NWIN, WIN)
    acc, den = _edge_pass(xw, av.reshape(NPAD), bv.reshape(NPAD), src3d, dst3d)
    out = _finalize(acc, den, bias.reshape(1, D))
    return out[:N]


# speedup vs baseline: 19.4986x; 19.4986x over previous
import dataclasses

import jax
import jax.numpy as jnp
from jax import lax
from jax.experimental import pallas as pl
from jax.experimental.pallas import tpu as pltpu
from jax.experimental.pallas import tpu_sc as plsc

N = 10000
D = 128
HD = D // 2
E = 320000
NPAD = 10240
NC, NS, L = 2, 16, 16
EPT = E // NS
WIN = 80
NWIN = EPT // WIN
RPT = NPAD // NS
BLK = 1024
NBLK = NPAD // BLK


def _proj_body(x_ref, w_ref, asrc_ref, adst_ref, xw_ref, av_ref, bv_ref):
    xw = jnp.dot(x_ref[...], w_ref[...], preferred_element_type=jnp.float32)
    xw_ref[...] = xw
    av_ref[...] = jnp.sum(xw * asrc_ref[...], axis=1, keepdims=True)
    bv_ref[...] = jnp.sum(xw * adst_ref[...], axis=1, keepdims=True)


def _project(x_pad, W, asrc_row, adst_row):
    return pl.pallas_call(
        _proj_body,
        grid=(NBLK,),
        in_specs=[
            pl.BlockSpec((BLK, D), lambda i: (i, 0)),
            pl.BlockSpec((D, D), lambda i: (0, 0)),
            pl.BlockSpec((1, D), lambda i: (0, 0)),
            pl.BlockSpec((1, D), lambda i: (0, 0)),
        ],
        out_specs=[
            pl.BlockSpec((BLK, D), lambda i: (i, 0)),
            pl.BlockSpec((BLK, 1), lambda i: (i, 0)),
            pl.BlockSpec((BLK, 1), lambda i: (i, 0)),
        ],
        out_shape=[
            jax.ShapeDtypeStruct((NPAD, D), jnp.float32),
            jax.ShapeDtypeStruct((NPAD, 1), jnp.float32),
            jax.ShapeDtypeStruct((NPAD, 1), jnp.float32),
        ],
    )(x_pad, W, asrc_row, adst_row)


def _edge_body(xwh_hbm, av_hbm, bv_hbm, src_hbm, dst_hbm,
               acc_hbm, den_hbm,
               asrc_t, adst_t, src_t, dst_t, rows_v, e_v, zden_v, acc_sh, den_sh):
    cid = lax.axis_index("c")
    sid = lax.axis_index("s")

    pltpu.sync_copy(av_hbm, asrc_t)
    pltpu.sync_copy(bv_hbm, adst_t)
    pltpu.sync_copy(src_hbm.at[sid], src_t)
    pltpu.sync_copy(dst_hbm.at[sid], dst_t)

    @pl.loop(0, WIN)
    def _(r):
        for k in range(HD // L):
            rows_v[r, pl.ds(k * L, L)] = jnp.zeros((L,), jnp.float32)

    @pl.loop(0, RPT, step=L)
    def _(i):
        zden_v[pl.ds(i, L)] = jnp.zeros((L,), jnp.float32)

    for k in range(RPT // WIN):
        pltpu.sync_copy(rows_v, acc_sh.at[pl.ds(sid * RPT + k * WIN, WIN)])
    pltpu.sync_copy(zden_v, den_sh.at[pl.ds(sid * RPT, RPT)])
    plsc.subcore_barrier()

    @pl.loop(0, NWIN)
    def _(j):
        for g in range(WIN // L):
            sl = pl.ds(g * L, L)
            s_idx = src_t[j, sl]
            d_idx = dst_t[j, sl]
            a_s = plsc.load_gather(asrc_t, [s_idx])
            a_d = plsc.load_gather(adst_t, [d_idx])
            al = a_s + a_d
            al = jnp.maximum(al, 0.2 * al)
            e_v[sl] = jnp.exp(al)

        pltpu.sync_copy(xwh_hbm.at[cid].at[src_t.at[j]], rows_v)

        @pl.loop(0, WIN)
        def _(r):
            bc = plsc.load_gather(e_v, [jnp.full((L,), r, jnp.int32)])
            for k in range(HD // L):
                sl = pl.ds(k * L, L)
                rows_v[r, sl] = rows_v[r, sl] * bc

        pltpu.sync_copy(rows_v, acc_sh.at[dst_t.at[j]], add=True)

        @pl.when(cid == 0)
        def _():
            pltpu.sync_copy(e_v, den_sh.at[dst_t.at[j]], add=True)

    plsc.subcore_barrier()

    pltpu.sync_copy(acc_sh.at[pl.ds(sid * RPT, RPT)],
                    acc_hbm.at[cid].at[pl.ds(sid * RPT, RPT)])

    @pl.when(cid == 0)
    def _():
        pltpu.sync_copy(den_sh.at[pl.ds(sid * RPT, RPT)],
                        den_hbm.at[pl.ds(sid * RPT, RPT)])


def _edge_pass(xw_halves, av, bv, src3d, dst3d):
    mesh = plsc.VectorSubcoreMesh(core_axis_name="c", subcore_axis_name="s")
    cp = pltpu.CompilerParams()
    if "needs_layout_passes" in pltpu.CompilerParams.__dataclass_fields__:
        cp = dataclasses.replace(cp, needs_layout_passes=False)
    if "use_tc_tiling_on_sc" in pltpu.CompilerParams.__dataclass_fields__:
        cp = dataclasses.replace(cp, use_tc_tiling_on_sc=False)
    f = pl.kernel(
        _edge_body,
        mesh=mesh,
        compiler_params=cp,
        out_type=[
            jax.ShapeDtypeStruct((NC, NPAD, HD), jnp.float32),
            jax.ShapeDtypeStruct((NPAD,), jnp.float32),
        ],
        scratch_types=[
            pltpu.VMEM((NPAD,), jnp.float32),
            pltpu.VMEM((NPAD,), jnp.float32),
            pltpu.VMEM((NWIN, WIN), jnp.int32),
            pltpu.VMEM((NWIN, WIN), jnp.int32),
            pltpu.VMEM((WIN, HD), jnp.float32),
            pltpu.VMEM((WIN,), jnp.float32),
            pltpu.VMEM((RPT,), jnp.float32),
            pltpu.VMEM_SHARED((NPAD, HD), jnp.float32),
            pltpu.VMEM_SHARED((NPAD,), jnp.float32),
        ],
    )
    return f(xw_halves, av, bv, src3d, dst3d)


def _final_body(a0_ref, a1_ref, d_ref, bias_ref, o_ref):
    dd = d_ref[...] + 1e-16
    lo = a0_ref[0] / dd + bias_ref[:, :HD]
    hi = a1_ref[0] / dd + bias_ref[:, HD:]
    v = jnp.concatenate([lo, hi], axis=1)
    o_ref[...] = jnp.where(v > 0, v, jnp.exp(jnp.minimum(v, 0.0)) - 1.0)


def _finalize(acc, den, bias_row):
    return pl.pallas_call(
        _final_body,
        grid=(NBLK,),
        in_specs=[
            pl.BlockSpec((1, BLK, HD), lambda i: (0, i, 0)),
            pl.BlockSpec((1, BLK, HD), lambda i: (1, i, 0)),
            pl.BlockSpec((BLK, 1), lambda i: (i, 0)),
            pl.BlockSpec((1, D), lambda i: (0, 0)),
        ],
        out_specs=pl.BlockSpec((BLK, D), lambda i: (i, 0)),
        out_shape=jax.ShapeDtypeStruct((NPAD, D), jnp.float32),
    )(acc, acc, den[:, None], bias_row)


def kernel(x, edge_index, W, att_src, att_dst, bias):
    x_pad = jnp.pad(x, ((0, NPAD - N), (0, 0)))
    asrc_row = att_src.reshape(1, D)
    adst_row = att_dst.reshape(1, D)
    xw, av, bv = _project(x_pad, W, asrc_row, adst_row)
    xw_halves = jnp.stack([xw[:, :HD], xw[:, HD:]])
    src3d = edge_index[0].reshape(NS, NWIN, WIN)
    dst3d = edge_index[1].reshape(NS, NWIN, WIN)
    acc, den = _edge_pass(xw_halves, av.reshape(NPAD), bv.reshape(NPAD),
                          src3d, dst3d)
    out = _finalize(acc, den, bias.reshape(1, D))
    return out[:N]
